# Initial kernel scaffold; baseline (speedup 1.0000x reference)
import jax
import jax.numpy as jnp
from jax.experimental import pallas as pl


def kernel(h_root, tasks, W1, b1, W2, b2):
    raise NotImplementedError("write your pallas kernel here")


# speedup vs baseline: 2.0103x; 2.0103x over previous
import functools

import jax
import jax.numpy as jnp
from jax import lax
from jax.experimental import pallas as pl
from jax.experimental.pallas import tpu as pltpu

N_TOKENS = 4096
D_MODEL = 2048
D_FF = 2048
N_TASKS = 8

TILE = 256
NT = N_TOKENS // TILE
NUM_ITEMS = NT + N_TASKS - 1


def _routing(tasks):
    t32 = tasks.astype(jnp.int32)
    sort_idx = jnp.argsort(t32).astype(jnp.int32)
    inv_idx = jnp.zeros((N_TOKENS,), jnp.int32).at[sort_idx].set(
        jnp.arange(N_TOKENS, dtype=jnp.int32))
    sorted_tasks = t32[sort_idx]
    offsets = jnp.searchsorted(
        sorted_tasks, jnp.arange(N_TASKS + 1, dtype=jnp.int32), side="left"
    ).astype(jnp.int32)
    st = sorted_tasks.reshape(NT, TILE)
    lo = st[:, 0]
    hi = st[:, -1]
    n_items = hi - lo + 1
    starts = jnp.concatenate(
        [jnp.zeros((1,), jnp.int32), jnp.cumsum(n_items, dtype=jnp.int32)])
    total = starts[NT]
    ii = jnp.arange(NUM_ITEMS, dtype=jnp.int32)
    t_of_i = jnp.clip(jnp.searchsorted(starts, ii, side="right") - 1, 0, NT - 1)
    e_of_i = jnp.clip(lo[t_of_i] + ii - starts[t_of_i], 0, N_TASKS - 1)
    pad = ii >= total
    tile_ids = jnp.where(pad, NT - 1, t_of_i)
    group_ids = jnp.where(pad, hi[NT - 1], e_of_i)
    return sort_idx, inv_idx, offsets, tile_ids, group_ids


NFF = 2
FFB = D_FF // NFF


def _mlp_item(ti_ref, gi_ref, off_ref,
              x_ref, w1_ref, b1_ref, w2_ref, b2_ref, out_ref, acc_ref):
    i = pl.program_id(0)
    j = pl.program_id(1)
    e = gi_ref[i]
    t = ti_ref[i]
    x = x_ref[...]
    b1e = b1_ref[pl.ds(e, 1), :]
    b1j = lax.select(j == 0, b1e[:, :FFB], b1e[:, FFB:])
    h = jnp.dot(x, w1_ref[0], preferred_element_type=jnp.float32)
    h = jnp.maximum(h + b1j, 0.0)
    part = jnp.dot(h, w2_ref[0], preferred_element_type=jnp.float32)

    @pl.when(j == 0)
    def _init():
        acc_ref[...] = part

    @pl.when(j != 0)
    def _acc():
        acc_ref[...] += part

    @pl.when(j == NFF - 1)
    def _emit():
        y = acc_ref[...] + b2_ref[pl.ds(e, 1), :]
        rows = t * TILE + lax.broadcasted_iota(jnp.int32, (TILE, 1), 0)
        mask = (rows >= off_ref[e]) & (rows < off_ref[e + 1])
        out_ref[...] = jnp.where(mask, y, out_ref[...])


def _grouped_mlp(x_sorted, W1, b1, W2, b2, offsets, tile_ids, group_ids):
    grid_spec = pltpu.PrefetchScalarGridSpec(
        num_scalar_prefetch=3,
        grid=(NUM_ITEMS, NFF),
        in_specs=[
            pl.BlockSpec((TILE, D_MODEL), lambda i, j, ti, gi, off: (ti[i], 0)),
            pl.BlockSpec((1, D_MODEL, FFB), lambda i, j, ti, gi, off: (gi[i], 0, j)),
            pl.BlockSpec((N_TASKS, D_FF), lambda i, j, ti, gi, off: (0, 0)),
            pl.BlockSpec((1, FFB, D_MODEL), lambda i, j, ti, gi, off: (gi[i], j, 0)),
            pl.BlockSpec((N_TASKS, D_MODEL), lambda i, j, ti, gi, off: (0, 0)),
        ],
        out_specs=pl.BlockSpec((TILE, D_MODEL), lambda i, j, ti, gi, off: (ti[i], 0)),
        scratch_shapes=[pltpu.VMEM((TILE, D_MODEL), jnp.float32)],
    )
    return pl.pallas_call(
        _mlp_item,
        grid_spec=grid_spec,
        out_shape=jax.ShapeDtypeStruct((N_TOKENS, D_MODEL), jnp.float32),
        compiler_params=pltpu.CompilerParams(
            dimension_semantics=("arbitrary", "arbitrary")),
    )(tile_ids, group_ids, offsets, x_sorted, W1, b1, W2, b2)


def kernel(h_root, tasks, W1, b1, W2, b2):
    sort_idx, inv_idx, offsets, tile_ids, group_ids = _routing(tasks)
    x_sorted = jnp.take(h_root, sort_idx, axis=0)
    y_sorted = _grouped_mlp(x_sorted, W1, b1, W2, b2,
                            offsets, tile_ids, group_ids)
    return jnp.take(y_sorted, inv_idx, axis=0)
